# Initial kernel scaffold; baseline (speedup 1.0000x reference)
import jax
import jax.numpy as jnp
from jax.experimental import pallas as pl


def kernel(features, params):
    raise NotImplementedError("write your pallas kernel here")


# speedup vs baseline: 9.9129x; 9.9129x over previous
import functools

import jax
import jax.numpy as jnp
import numpy as np
from jax.experimental import pallas as pl
from jax.experimental.pallas import tpu as pltpu

B = 8
FEAT_DIM = 384
GH = GW = 37
N = 2048
N_STEPS = 8
K = 6
HID = 128
STATE_DIM = 16
RB = 512
MB = 512


def _dot(a, b, prec=jax.lax.Precision.DEFAULT):
    return jax.lax.dot_general(a, b, (((1,), (0,)), ((), ())),
                               precision=prec,
                               preferred_element_type=jnp.float32)


def _init_kernel(s_ref, img_ref, w1_ref, b1_ref, w2_ref, b2_ref, w3_ref,
                 b3_ref, out_ref):
    samp = _dot(s_ref[...], img_ref[0], jax.lax.Precision.HIGHEST)
    h = jnp.maximum(_dot(samp, w1_ref[...]) + b1_ref[...], 0.0)
    h = jnp.maximum(_dot(h, w2_ref[...]) + b2_ref[...], 0.0)
    out_ref[0] = _dot(h, w3_ref[...]) + b3_ref[...]


def _run_init(s_mat, img_f, params):
    full = lambda shape: pl.BlockSpec(shape, lambda b: (0,) * len(shape))
    return pl.pallas_call(
        _init_kernel,
        grid=(B,),
        in_specs=[
            full((N, GH * GW)),
            pl.BlockSpec((1, GH * GW, FEAT_DIM), lambda b: (b, 0, 0)),
            full((FEAT_DIM, 2 * HID)), full((1, 2 * HID)),
            full((2 * HID, HID)), full((1, HID)),
            full((HID, STATE_DIM)), full((1, STATE_DIM)),
        ],
        out_specs=pl.BlockSpec((1, N, STATE_DIM), lambda b: (b, 0, 0)),
        out_shape=jax.ShapeDtypeStruct((B, N, STATE_DIM), jnp.float32),
        compiler_params=pltpu.CompilerParams(
            dimension_semantics=("arbitrary",)),
    )(s_mat, img_f,
      params['init_W1'], params['init_b1'].reshape(1, -1),
      params['init_W2'], params['init_b2'].reshape(1, -1),
      params['init_W3'], params['init_b3'].reshape(1, -1))


def _topk_kernel(posc_ref, posr_ref, out_ref):
    pc = posc_ref[0]
    pr = posr_ref[0]
    xc, yc, zc = pc[:, 0:1], pc[:, 1:2], pc[:, 2:3]
    xr, yr, zr = pr[0:1, :], pr[1:2, :], pr[2:3, :]
    sqc = xc * xc + yc * yc + zc * zc
    sqr = xr * xr + yr * yr + zr * zr
    dot = _dot(pc, pr)
    d2 = (sqc + sqr) - 2.0 * dot
    iota = jax.lax.broadcasted_iota(jnp.int32, (RB, N), 1)
    vals = d2
    for t in range(K + 1):
        m = jnp.min(vals, axis=1, keepdims=True)
        idxm = jnp.where(vals == m, iota, N)
        sel = jnp.min(idxm, axis=1, keepdims=True)
        out_ref[0, :, pl.ds(t, 1)] = sel
        if t < K:
            vals = jnp.where(idxm == sel, jnp.inf, vals)


def _run_topk(posc, posr):
    return pl.pallas_call(
        _topk_kernel,
        grid=(B, N // RB),
        in_specs=[
            pl.BlockSpec((1, RB, 8), lambda b, r: (b, r, 0)),
            pl.BlockSpec((1, 8, N), lambda b, r: (b, 0, 0)),
        ],
        out_specs=pl.BlockSpec((1, RB, 8), lambda b, r: (b, r, 0)),
        out_shape=jax.ShapeDtypeStruct((B, N, 8), jnp.int32),
        compiler_params=pltpu.CompilerParams(
            dimension_semantics=("arbitrary", "arbitrary")),
    )(posc, posr)


def _mlp_kernel(st_ref, g_ref, w1_ref, b1_ref, w2_ref, b2_ref,
                u1_ref, ub1_ref, u2_ref, ub2_ref, step_ref, out_ref):
    st = st_ref[...]
    pin = jnp.concatenate([st, g_ref[...]], axis=1)
    h = jnp.maximum(_dot(pin, w1_ref[...]) + b1_ref[...], 0.0)
    h = jnp.maximum(_dot(h, w2_ref[...]) + b2_ref[...], 0.0)
    h = jnp.maximum(_dot(h, u1_ref[...]) + ub1_ref[...], 0.0)
    delta = _dot(h, u2_ref[...]) + ub2_ref[...]
    out_ref[...] = st + step_ref[0, 0] * delta


def _run_mlp(st2d, g2d, w1, b1, w2, b2, u1, ub1, u2, ub2, step):
    full = lambda shape: pl.BlockSpec(shape, lambda i: (0,) * len(shape))
    return pl.pallas_call(
        _mlp_kernel,
        grid=(B * N // MB,),
        in_specs=[
            pl.BlockSpec((MB, STATE_DIM), lambda i: (i, 0)),
            pl.BlockSpec((MB, K * STATE_DIM), lambda i: (i, 0)),
            full(((K + 1) * STATE_DIM, 2 * HID)),
            full((1, 2 * HID)), full((2 * HID, HID)), full((1, HID)),
            full((HID, HID)), full((1, HID)), full((HID, STATE_DIM)),
            full((1, STATE_DIM)), full((1, 1)),
        ],
        out_specs=pl.BlockSpec((MB, STATE_DIM), lambda i: (i, 0)),
        out_shape=jax.ShapeDtypeStruct((B * N, STATE_DIM), jnp.float32),
        compiler_params=pltpu.CompilerParams(
            dimension_semantics=("arbitrary",)),
    )(st2d, g2d, w1, b1, w2, b2, u1, ub1, u2, ub2, step)


def _decode_kernel(st_ref, out_ref):
    st = st_ref[...]
    out_ref[:, 0:3] = st[:, 0:3]
    sc_in = jnp.clip(st[:, 3:6], -10.0, 20.0) + 1.0
    sp = jnp.maximum(sc_in, 0.0) + jnp.log1p(jnp.exp(-jnp.abs(sc_in)))
    out_ref[:, 3:6] = jnp.clip(sp * 0.15, 1e-6, 2.0)
    a1x, a1y, a1z = st[:, 6:7], st[:, 7:8], st[:, 8:9]
    a2x, a2y, a2z = st[:, 9:10], st[:, 10:11], st[:, 11:12]
    n1 = jnp.sqrt(a1x * a1x + a1y * a1y + a1z * a1z) + 1e-8
    b1x, b1y, b1z = a1x / n1, a1y / n1, a1z / n1
    dot = b1x * a2x + b1y * a2y + b1z * a2z
    px, py, pz = a2x - dot * b1x, a2y - dot * b1y, a2z - dot * b1z
    n2 = jnp.sqrt(px * px + py * py + pz * pz) + 1e-8
    b2x, b2y, b2z = px / n2, py / n2, pz / n2
    b3x = b1y * b2z - b1z * b2y
    b3y = b1z * b2x - b1x * b2z
    b3z = b1x * b2y - b1y * b2x
    w = 0.5 * jnp.sqrt(jnp.clip(1.0 + b1x + b2y + b3z, 1e-8))
    out_ref[:, 6:7] = w
    out_ref[:, 7:8] = (b3y - b2z) / (4.0 * w)
    out_ref[:, 8:9] = (b1z - b3x) / (4.0 * w)
    out_ref[:, 9:10] = (b2x - b1y) / (4.0 * w)
    out_ref[:, 10:13] = jax.nn.sigmoid(st[:, 12:15])
    out_ref[:, 13:14] = jax.nn.sigmoid(st[:, 15:16])


def _run_decode(st2d):
    return pl.pallas_call(
        _decode_kernel,
        grid=(B * N // MB,),
        in_specs=[pl.BlockSpec((MB, STATE_DIM), lambda i: (i, 0))],
        out_specs=pl.BlockSpec((MB, 14), lambda i: (i, 0)),
        out_shape=jax.ShapeDtypeStruct((B * N, 14), jnp.float32),
        compiler_params=pltpu.CompilerParams(
            dimension_semantics=("arbitrary",)),
    )(st2d)


def _sample_matrix():
    i = jnp.arange(N, dtype=jnp.float32)
    ga = np.pi * (3.0 - np.sqrt(5.0))
    r = jnp.sqrt((i + 0.5) / N)
    th = i * ga
    sx = r * jnp.cos(th) * 0.95
    sy = r * jnp.sin(th) * 0.95
    ix = (sx + 1.0) * 0.5 * (GW - 1)
    iy = (sy + 1.0) * 0.5 * (GH - 1)
    x0 = jnp.floor(ix)
    y0 = jnp.floor(iy)
    wx1 = ix - x0
    wy1 = iy - y0
    x0c = jnp.clip(x0, 0, GW - 1).astype(jnp.int32)
    x1c = jnp.clip(x0 + 1, 0, GW - 1).astype(jnp.int32)
    y0c = jnp.clip(y0, 0, GH - 1).astype(jnp.int32)
    y1c = jnp.clip(y0 + 1, 0, GH - 1).astype(jnp.int32)
    hw = GH * GW
    s = (jax.nn.one_hot(y0c * GW + x0c, hw) * ((1 - wy1) * (1 - wx1))[:, None]
         + jax.nn.one_hot(y0c * GW + x1c, hw) * ((1 - wy1) * wx1)[:, None]
         + jax.nn.one_hot(y1c * GW + x0c, hw) * (wy1 * (1 - wx1))[:, None]
         + jax.nn.one_hot(y1c * GW + x1c, hw) * (wy1 * wx1)[:, None])
    return s, sx, sy


def _gather_neighbors(state2d, flat_idx):
    return state2d[flat_idx]


@functools.partial(jax.jit, static_argnames=())
def kernel(features, params):
    s_mat, sx, sy = _sample_matrix()
    img_f = jnp.transpose(features.reshape(B, FEAT_DIM, GH * GW), (0, 2, 1))
    st0 = _run_init(s_mat, img_f, params)

    ch0 = sx[None, :] + st0[..., 0] * 0.15
    ch1 = sy[None, :] + st0[..., 1] * 0.15
    base_z = jnp.broadcast_to(params['depth_offset'], (B, N))
    state = jnp.concatenate(
        [ch0[..., None], ch1[..., None], base_z[..., None], st0[..., 3:]],
        axis=-1)

    w1 = params['perc_W1']
    b1 = params['perc_b1'].reshape(1, -1)
    w2, b2 = params['perc_W2'], params['perc_b2'].reshape(1, -1)
    u1, ub1 = params['upd_W1'], params['upd_b1'].reshape(1, -1)
    u2, ub2 = params['upd_W2'], params['upd_b2'].reshape(1, -1)
    step = params['step_size'].reshape(1, 1)
    boff = (jnp.arange(B, dtype=jnp.int32) * N)[:, None, None]

    for _ in range(N_STEPS):
        pos = state[..., :3]
        posc = jnp.concatenate(
            [pos, jnp.zeros((B, N, 5), jnp.float32)], axis=-1)
        posr = jnp.transpose(posc, (0, 2, 1))
        nbr8 = _run_topk(posc, posr)
        flat_idx = (nbr8[:, :, 1:K + 1] + boff).reshape(-1)
        state2d = state.reshape(B * N, STATE_DIM)
        g2d = _gather_neighbors(state2d, flat_idx).reshape(B * N, K * STATE_DIM)
        state2d = _run_mlp(state2d, g2d, w1, b1, w2, b2, u1, ub1, u2,
                           ub2, step)
        state = state2d.reshape(B, N, STATE_DIM)

    out = _run_decode(state.reshape(B * N, STATE_DIM))
    return out.reshape(B, N, 14)


# speedup vs baseline: 15.5707x; 1.5707x over previous
import functools

import jax
import jax.numpy as jnp
import numpy as np
from jax.experimental import pallas as pl
from jax.experimental.pallas import tpu as pltpu
from jax.experimental.pallas import tpu_sc as plsc

B = 8
FEAT_DIM = 384
GH = GW = 37
N = 2048
N_STEPS = 8
K = 6
HID = 128
STATE_DIM = 16
RB = 512
MB = 512


def _dot(a, b, prec=jax.lax.Precision.DEFAULT):
    return jax.lax.dot_general(a, b, (((1,), (0,)), ((), ())),
                               precision=prec,
                               preferred_element_type=jnp.float32)


def _init_kernel(s_ref, img_ref, w1_ref, b1_ref, w2_ref, b2_ref, w3_ref,
                 b3_ref, out_ref):
    samp = _dot(s_ref[...], img_ref[0], jax.lax.Precision.HIGHEST)
    h = jnp.maximum(_dot(samp, w1_ref[...]) + b1_ref[...], 0.0)
    h = jnp.maximum(_dot(h, w2_ref[...]) + b2_ref[...], 0.0)
    out_ref[0] = _dot(h, w3_ref[...]) + b3_ref[...]


def _run_init(s_mat, img_f, params):
    full = lambda shape: pl.BlockSpec(shape, lambda b: (0,) * len(shape))
    return pl.pallas_call(
        _init_kernel,
        grid=(B,),
        in_specs=[
            full((N, GH * GW)),
            pl.BlockSpec((1, GH * GW, FEAT_DIM), lambda b: (b, 0, 0)),
            full((FEAT_DIM, 2 * HID)), full((1, 2 * HID)),
            full((2 * HID, HID)), full((1, HID)),
            full((HID, STATE_DIM)), full((1, STATE_DIM)),
        ],
        out_specs=pl.BlockSpec((1, N, STATE_DIM), lambda b: (b, 0, 0)),
        out_shape=jax.ShapeDtypeStruct((B, N, STATE_DIM), jnp.float32),
        compiler_params=pltpu.CompilerParams(
            dimension_semantics=("arbitrary",)),
    )(s_mat, img_f,
      params['init_W1'], params['init_b1'].reshape(1, -1),
      params['init_W2'], params['init_b2'].reshape(1, -1),
      params['init_W3'], params['init_b3'].reshape(1, -1))


def _topk_kernel(posc_ref, posr_ref, out_ref):
    pc = posc_ref[0]
    pr = posr_ref[0]
    xc, yc, zc = pc[:, 0:1], pc[:, 1:2], pc[:, 2:3]
    xr, yr, zr = pr[0:1, :], pr[1:2, :], pr[2:3, :]
    sqc = xc * xc + yc * yc + zc * zc
    sqr = xr * xr + yr * yr + zr * zr
    dot = _dot(pc, pr)
    d2 = (sqc + sqr) - 2.0 * dot
    iota = jax.lax.broadcasted_iota(jnp.int32, (RB, N), 1)
    boff = pl.program_id(0) * N
    vals = d2
    for t in range(K + 1):
        m = jnp.min(vals, axis=1, keepdims=True)
        idxm = jnp.where(vals == m, iota, N)
        sel = jnp.min(idxm, axis=1, keepdims=True)
        out_ref[0, :, pl.ds(t - 1 if t > 0 else 6, 1)] = sel + boff
        if t < K:
            vals = jnp.where(idxm == sel, jnp.inf, vals)


def _run_topk(posc, posr):
    return pl.pallas_call(
        _topk_kernel,
        grid=(B, N // RB),
        in_specs=[
            pl.BlockSpec((1, RB, 8), lambda b, r: (b, r, 0)),
            pl.BlockSpec((1, 8, N), lambda b, r: (b, 0, 0)),
        ],
        out_specs=pl.BlockSpec((1, RB, 8), lambda b, r: (b, r, 0)),
        out_shape=jax.ShapeDtypeStruct((B, N, 8), jnp.int32),
        compiler_params=pltpu.CompilerParams(
            dimension_semantics=("arbitrary", "arbitrary")),
    )(posc, posr)


def _mlp_kernel(st_ref, g_ref, w1_ref, b1_ref, w2_ref, b2_ref,
                u1_ref, ub1_ref, u2_ref, ub2_ref, step_ref, out_ref, pad_ref):
    st = st_ref[...]
    g = g_ref[...]
    pin = jnp.concatenate(
        [st] + [g[:, k * _PD:k * _PD + STATE_DIM] for k in range(K)], axis=1)
    h = jnp.maximum(_dot(pin, w1_ref[...]) + b1_ref[...], 0.0)
    h = jnp.maximum(_dot(h, w2_ref[...]) + b2_ref[...], 0.0)
    h = jnp.maximum(_dot(h, u1_ref[...]) + ub1_ref[...], 0.0)
    delta = _dot(h, u2_ref[...]) + ub2_ref[...]
    new = st + step_ref[0, 0] * delta
    out_ref[...] = new
    pad_ref[...] = jnp.concatenate(
        [new, jnp.zeros((MB, _PD - STATE_DIM), jnp.float32)], axis=1)


def _run_mlp(st2d, g2d, w1, b1, w2, b2, u1, ub1, u2, ub2, step):
    full = lambda shape: pl.BlockSpec(shape, lambda i: (0,) * len(shape))
    return pl.pallas_call(
        _mlp_kernel,
        grid=(B * N // MB,),
        in_specs=[
            pl.BlockSpec((MB, STATE_DIM), lambda i: (i, 0)),
            pl.BlockSpec((MB, K * _PD), lambda i: (i, 0)),
            full(((K + 1) * STATE_DIM, 2 * HID)),
            full((1, 2 * HID)), full((2 * HID, HID)), full((1, HID)),
            full((HID, HID)), full((1, HID)), full((HID, STATE_DIM)),
            full((1, STATE_DIM)), full((1, 1)),
        ],
        out_specs=[pl.BlockSpec((MB, STATE_DIM), lambda i: (i, 0)),
                   pl.BlockSpec((MB, _PD), lambda i: (i, 0))],
        out_shape=[jax.ShapeDtypeStruct((B * N, STATE_DIM), jnp.float32),
                   jax.ShapeDtypeStruct((B * N, _PD), jnp.float32)],
        compiler_params=pltpu.CompilerParams(
            dimension_semantics=("arbitrary",)),
    )(st2d, g2d, w1, b1, w2, b2, u1, ub1, u2, ub2, step)


def _decode_kernel(st_ref, out_ref):
    st = st_ref[...]
    out_ref[:, 0:3] = st[:, 0:3]
    sc_in = jnp.clip(st[:, 3:6], -10.0, 20.0) + 1.0
    sp = jnp.maximum(sc_in, 0.0) + jnp.log1p(jnp.exp(-jnp.abs(sc_in)))
    out_ref[:, 3:6] = jnp.clip(sp * 0.15, 1e-6, 2.0)
    a1x, a1y, a1z = st[:, 6:7], st[:, 7:8], st[:, 8:9]
    a2x, a2y, a2z = st[:, 9:10], st[:, 10:11], st[:, 11:12]
    n1 = jnp.sqrt(a1x * a1x + a1y * a1y + a1z * a1z) + 1e-8
    b1x, b1y, b1z = a1x / n1, a1y / n1, a1z / n1
    dot = b1x * a2x + b1y * a2y + b1z * a2z
    px, py, pz = a2x - dot * b1x, a2y - dot * b1y, a2z - dot * b1z
    n2 = jnp.sqrt(px * px + py * py + pz * pz) + 1e-8
    b2x, b2y, b2z = px / n2, py / n2, pz / n2
    b3x = b1y * b2z - b1z * b2y
    b3y = b1z * b2x - b1x * b2z
    b3z = b1x * b2y - b1y * b2x
    w = 0.5 * jnp.sqrt(jnp.clip(1.0 + b1x + b2y + b3z, 1e-8))
    out_ref[:, 6:7] = w
    out_ref[:, 7:8] = (b3y - b2z) / (4.0 * w)
    out_ref[:, 8:9] = (b1z - b3x) / (4.0 * w)
    out_ref[:, 9:10] = (b2x - b1y) / (4.0 * w)
    out_ref[:, 10:13] = jax.nn.sigmoid(st[:, 12:15])
    out_ref[:, 13:14] = jax.nn.sigmoid(st[:, 15:16])


def _run_decode(st2d):
    return pl.pallas_call(
        _decode_kernel,
        grid=(B * N // MB,),
        in_specs=[pl.BlockSpec((MB, STATE_DIM), lambda i: (i, 0))],
        out_specs=pl.BlockSpec((MB, 14), lambda i: (i, 0)),
        out_shape=jax.ShapeDtypeStruct((B * N, 14), jnp.float32),
        compiler_params=pltpu.CompilerParams(
            dimension_semantics=("arbitrary",)),
    )(st2d)


def _sample_matrix():
    i = jnp.arange(N, dtype=jnp.float32)
    ga = np.pi * (3.0 - np.sqrt(5.0))
    r = jnp.sqrt((i + 0.5) / N)
    th = i * ga
    sx = r * jnp.cos(th) * 0.95
    sy = r * jnp.sin(th) * 0.95
    ix = (sx + 1.0) * 0.5 * (GW - 1)
    iy = (sy + 1.0) * 0.5 * (GH - 1)
    x0 = jnp.floor(ix)
    y0 = jnp.floor(iy)
    wx1 = ix - x0
    wy1 = iy - y0
    x0c = jnp.clip(x0, 0, GW - 1).astype(jnp.int32)
    x1c = jnp.clip(x0 + 1, 0, GW - 1).astype(jnp.int32)
    y0c = jnp.clip(y0, 0, GH - 1).astype(jnp.int32)
    y1c = jnp.clip(y0 + 1, 0, GH - 1).astype(jnp.int32)
    hw = GH * GW
    s = (jax.nn.one_hot(y0c * GW + x0c, hw) * ((1 - wy1) * (1 - wx1))[:, None]
         + jax.nn.one_hot(y0c * GW + x1c, hw) * ((1 - wy1) * wx1)[:, None]
         + jax.nn.one_hot(y1c * GW + x0c, hw) * (wy1 * (1 - wx1))[:, None]
         + jax.nn.one_hot(y1c * GW + x1c, hw) * (wy1 * wx1)[:, None])
    return s, sx, sy


_NUMI = B * N * K


_NW = 32
_BPW = _NUMI // _NW
_GCH = 4
_GROW = _BPW // _GCH
_PD = 128


def _sc_gather(statepad, idx_flat):
    mesh = plsc.VectorSubcoreMesh(core_axis_name="c", subcore_axis_name="s")

    @functools.partial(
        pl.kernel,
        out_type=jax.ShapeDtypeStruct((_NUMI, _PD), jnp.float32),
        mesh=mesh,
        scratch_types=[
            pltpu.VMEM((_BPW,), jnp.int32),
            pltpu.VMEM((_GROW, _PD), jnp.float32),
            pltpu.SemaphoreType.DMA,
        ])
    def k(table_hbm, idx_hbm, out_hbm, idx_v, rows_v, sem):
        wid = jax.lax.axis_index("s") * 2 + jax.lax.axis_index("c")
        base = wid * _BPW
        pltpu.sync_copy(idx_hbm.at[pl.ds(base, _BPW)], idx_v)

        @pl.loop(0, _GCH)
        def _(c):
            off = c * _GROW
            pltpu.async_copy(table_hbm.at[idx_v.at[pl.ds(off, _GROW)]],
                             rows_v, sem).wait()
            pltpu.sync_copy(rows_v, out_hbm.at[pl.ds(base + off, _GROW)])

    return k(statepad, idx_flat)


def _pad_kernel(st_ref, out_ref):
    out_ref[...] = jnp.concatenate(
        [st_ref[...], jnp.zeros((MB, _PD - STATE_DIM), jnp.float32)], axis=1)


def _run_pad(st2d):
    return pl.pallas_call(
        _pad_kernel,
        grid=(B * N // MB,),
        in_specs=[pl.BlockSpec((MB, STATE_DIM), lambda i: (i, 0))],
        out_specs=pl.BlockSpec((MB, _PD), lambda i: (i, 0)),
        out_shape=jax.ShapeDtypeStruct((B * N, _PD), jnp.float32),
        compiler_params=pltpu.CompilerParams(
            dimension_semantics=("arbitrary",)),
    )(st2d)


@functools.partial(jax.jit, static_argnames=())
def kernel(features, params):
    s_mat, sx, sy = _sample_matrix()
    img_f = jnp.transpose(features.reshape(B, FEAT_DIM, GH * GW), (0, 2, 1))
    st0 = _run_init(s_mat, img_f, params)

    ch0 = sx[None, :] + st0[..., 0] * 0.15
    ch1 = sy[None, :] + st0[..., 1] * 0.15
    base_z = jnp.broadcast_to(params['depth_offset'], (B, N))
    state = jnp.concatenate(
        [ch0[..., None], ch1[..., None], base_z[..., None], st0[..., 3:]],
        axis=-1)

    w1 = params['perc_W1']
    b1 = params['perc_b1'].reshape(1, -1)
    w2, b2 = params['perc_W2'], params['perc_b2'].reshape(1, -1)
    u1, ub1 = params['upd_W1'], params['upd_b1'].reshape(1, -1)
    u2, ub2 = params['upd_W2'], params['upd_b2'].reshape(1, -1)
    step = params['step_size'].reshape(1, 1)

    state2d = state.reshape(B * N, STATE_DIM)
    statepad = _run_pad(state2d)
    for _ in range(N_STEPS):
        pos = state[..., :3]
        posc = jnp.concatenate(
            [pos, jnp.zeros((B, N, 5), jnp.float32)], axis=-1)
        posr = jnp.transpose(posc, (0, 2, 1))
        nbr8 = _run_topk(posc, posr)
        flat_idx = nbr8[:, :, :K].reshape(_NUMI)
        g2d = _sc_gather(statepad, flat_idx).reshape(B * N, K * _PD)
        state2d, statepad = _run_mlp(state2d, g2d, w1, b1, w2, b2, u1, ub1,
                                     u2, ub2, step)
        state = state2d.reshape(B, N, STATE_DIM)

    out = _run_decode(state.reshape(B * N, STATE_DIM))
    return out.reshape(B, N, 14)


# speedup vs baseline: 18.0814x; 1.1612x over previous
import functools

import jax
import jax.numpy as jnp
import numpy as np
from jax.experimental import pallas as pl
from jax.experimental.pallas import tpu as pltpu
from jax.experimental.pallas import tpu_sc as plsc

B = 8
FEAT_DIM = 384
GH = GW = 37
N = 2048
N_STEPS = 8
K = 6
HID = 128
STATE_DIM = 16
RB = 512
MB = 2048


def _dot(a, b, prec=jax.lax.Precision.DEFAULT):
    return jax.lax.dot_general(a, b, (((1,), (0,)), ((), ())),
                               precision=prec,
                               preferred_element_type=jnp.float32)


def _init_kernel(s_ref, img_ref, w1_ref, b1_ref, w2_ref, b2_ref, w3_ref,
                 b3_ref, out_ref):
    samp = _dot(s_ref[...], img_ref[0], jax.lax.Precision.HIGHEST)
    h = jnp.maximum(_dot(samp, w1_ref[...]) + b1_ref[...], 0.0)
    h = jnp.maximum(_dot(h, w2_ref[...]) + b2_ref[...], 0.0)
    out_ref[0] = _dot(h, w3_ref[...]) + b3_ref[...]


def _run_init(s_mat, img_f, params):
    full = lambda shape: pl.BlockSpec(shape, lambda b: (0,) * len(shape))
    return pl.pallas_call(
        _init_kernel,
        grid=(B,),
        in_specs=[
            full((N, GH * GW)),
            pl.BlockSpec((1, GH * GW, FEAT_DIM), lambda b: (b, 0, 0)),
            full((FEAT_DIM, 2 * HID)), full((1, 2 * HID)),
            full((2 * HID, HID)), full((1, HID)),
            full((HID, STATE_DIM)), full((1, STATE_DIM)),
        ],
        out_specs=pl.BlockSpec((1, N, STATE_DIM), lambda b: (b, 0, 0)),
        out_shape=jax.ShapeDtypeStruct((B, N, STATE_DIM), jnp.float32),
        compiler_params=pltpu.CompilerParams(
            dimension_semantics=("parallel",)),
    )(s_mat, img_f,
      params['init_W1'], params['init_b1'].reshape(1, -1),
      params['init_W2'], params['init_b2'].reshape(1, -1),
      params['init_W3'], params['init_b3'].reshape(1, -1))


def _topk_kernel(posc_ref, posr_ref, out_ref):
    pc = posc_ref[0]
    pr = posr_ref[0]
    xc, yc, zc = pc[:, 0:1], pc[:, 1:2], pc[:, 2:3]
    xr, yr, zr = pr[0:1, :], pr[1:2, :], pr[2:3, :]
    sqc = xc * xc + yc * yc + zc * zc
    sqr = xr * xr + yr * yr + zr * zr
    dot = _dot(pc, pr)
    d2 = (sqc + sqr) - 2.0 * dot
    iota = jax.lax.broadcasted_iota(jnp.int32, (RB, N), 1)
    boff = pl.program_id(0) * N
    vals = d2
    for t in range(K + 1):
        m = jnp.min(vals, axis=1, keepdims=True)
        idxm = jnp.where(vals == m, iota, N)
        sel = jnp.min(idxm, axis=1, keepdims=True)
        out_ref[0, :, pl.ds(t - 1 if t > 0 else 6, 1)] = sel + boff
        if t < K:
            vals = jnp.where(idxm == sel, jnp.inf, vals)


def _run_topk(posc, posr):
    return pl.pallas_call(
        _topk_kernel,
        grid=(B, N // RB),
        in_specs=[
            pl.BlockSpec((1, RB, 8), lambda b, r: (b, r, 0)),
            pl.BlockSpec((1, 8, N), lambda b, r: (b, 0, 0)),
        ],
        out_specs=pl.BlockSpec((1, RB, 8), lambda b, r: (b, r, 0)),
        out_shape=jax.ShapeDtypeStruct((B, N, 8), jnp.int32),
        compiler_params=pltpu.CompilerParams(
            dimension_semantics=("parallel", "parallel")),
    )(posc, posr)


def _mlp_kernel(st_ref, g_ref, w1_ref, b1_ref, w2_ref, b2_ref,
                u1_ref, ub1_ref, u2_ref, ub2_ref, step_ref, out_ref, pad_ref):
    st = st_ref[...]
    g = g_ref[...]
    pin = jnp.concatenate(
        [st] + [g[k, :, :STATE_DIM] for k in range(K)], axis=1)
    h = jnp.maximum(_dot(pin, w1_ref[...]) + b1_ref[...], 0.0)
    h = jnp.maximum(_dot(h, w2_ref[...]) + b2_ref[...], 0.0)
    h = jnp.maximum(_dot(h, u1_ref[...]) + ub1_ref[...], 0.0)
    delta = _dot(h, u2_ref[...]) + ub2_ref[...]
    new = st + step_ref[0, 0] * delta
    out_ref[...] = new
    pad_ref[...] = jnp.concatenate(
        [new, jnp.zeros((MB, _PD - STATE_DIM), jnp.float32)], axis=1)


def _run_mlp(st2d, g2d, w1, b1, w2, b2, u1, ub1, u2, ub2, step):
    full = lambda shape: pl.BlockSpec(shape, lambda i: (0,) * len(shape))
    return pl.pallas_call(
        _mlp_kernel,
        grid=(B * N // MB,),
        in_specs=[
            pl.BlockSpec((MB, STATE_DIM), lambda i: (i, 0)),
            pl.BlockSpec((K, MB, _PD), lambda i: (0, i, 0)),
            full(((K + 1) * STATE_DIM, 2 * HID)),
            full((1, 2 * HID)), full((2 * HID, HID)), full((1, HID)),
            full((HID, HID)), full((1, HID)), full((HID, STATE_DIM)),
            full((1, STATE_DIM)), full((1, 1)),
        ],
        out_specs=[pl.BlockSpec((MB, STATE_DIM), lambda i: (i, 0)),
                   pl.BlockSpec((MB, _PD), lambda i: (i, 0))],
        out_shape=[jax.ShapeDtypeStruct((B * N, STATE_DIM), jnp.float32),
                   jax.ShapeDtypeStruct((B * N, _PD), jnp.float32)],
        compiler_params=pltpu.CompilerParams(
            dimension_semantics=("parallel",)),
    )(st2d, g2d, w1, b1, w2, b2, u1, ub1, u2, ub2, step)


def _decode_kernel(st_ref, out_ref):
    st = st_ref[...]
    out_ref[:, 0:3] = st[:, 0:3]
    sc_in = jnp.clip(st[:, 3:6], -10.0, 20.0) + 1.0
    sp = jnp.maximum(sc_in, 0.0) + jnp.log1p(jnp.exp(-jnp.abs(sc_in)))
    out_ref[:, 3:6] = jnp.clip(sp * 0.15, 1e-6, 2.0)
    a1x, a1y, a1z = st[:, 6:7], st[:, 7:8], st[:, 8:9]
    a2x, a2y, a2z = st[:, 9:10], st[:, 10:11], st[:, 11:12]
    n1 = jnp.sqrt(a1x * a1x + a1y * a1y + a1z * a1z) + 1e-8
    b1x, b1y, b1z = a1x / n1, a1y / n1, a1z / n1
    dot = b1x * a2x + b1y * a2y + b1z * a2z
    px, py, pz = a2x - dot * b1x, a2y - dot * b1y, a2z - dot * b1z
    n2 = jnp.sqrt(px * px + py * py + pz * pz) + 1e-8
    b2x, b2y, b2z = px / n2, py / n2, pz / n2
    b3x = b1y * b2z - b1z * b2y
    b3y = b1z * b2x - b1x * b2z
    b3z = b1x * b2y - b1y * b2x
    w = 0.5 * jnp.sqrt(jnp.clip(1.0 + b1x + b2y + b3z, 1e-8))
    out_ref[:, 6:7] = w
    out_ref[:, 7:8] = (b3y - b2z) / (4.0 * w)
    out_ref[:, 8:9] = (b1z - b3x) / (4.0 * w)
    out_ref[:, 9:10] = (b2x - b1y) / (4.0 * w)
    out_ref[:, 10:13] = jax.nn.sigmoid(st[:, 12:15])
    out_ref[:, 13:14] = jax.nn.sigmoid(st[:, 15:16])


def _run_decode(st2d):
    return pl.pallas_call(
        _decode_kernel,
        grid=(B * N // MB,),
        in_specs=[pl.BlockSpec((MB, STATE_DIM), lambda i: (i, 0))],
        out_specs=pl.BlockSpec((MB, 14), lambda i: (i, 0)),
        out_shape=jax.ShapeDtypeStruct((B * N, 14), jnp.float32),
        compiler_params=pltpu.CompilerParams(
            dimension_semantics=("parallel",)),
    )(st2d)


def _sample_matrix():
    i = jnp.arange(N, dtype=jnp.float32)
    ga = np.pi * (3.0 - np.sqrt(5.0))
    r = jnp.sqrt((i + 0.5) / N)
    th = i * ga
    sx = r * jnp.cos(th) * 0.95
    sy = r * jnp.sin(th) * 0.95
    ix = (sx + 1.0) * 0.5 * (GW - 1)
    iy = (sy + 1.0) * 0.5 * (GH - 1)
    x0 = jnp.floor(ix)
    y0 = jnp.floor(iy)
    wx1 = ix - x0
    wy1 = iy - y0
    x0c = jnp.clip(x0, 0, GW - 1).astype(jnp.int32)
    x1c = jnp.clip(x0 + 1, 0, GW - 1).astype(jnp.int32)
    y0c = jnp.clip(y0, 0, GH - 1).astype(jnp.int32)
    y1c = jnp.clip(y0 + 1, 0, GH - 1).astype(jnp.int32)
    hw = GH * GW
    s = (jax.nn.one_hot(y0c * GW + x0c, hw) * ((1 - wy1) * (1 - wx1))[:, None]
         + jax.nn.one_hot(y0c * GW + x1c, hw) * ((1 - wy1) * wx1)[:, None]
         + jax.nn.one_hot(y1c * GW + x0c, hw) * (wy1 * (1 - wx1))[:, None]
         + jax.nn.one_hot(y1c * GW + x1c, hw) * (wy1 * wx1)[:, None])
    return s, sx, sy


_NUMI = B * N * K


_NW = 32
_BPW = _NUMI // _NW
_GCH = 4
_GROW = _BPW // _GCH
_PD = 128


def _sc_gather(statepad, idx_flat):
    mesh = plsc.VectorSubcoreMesh(core_axis_name="c", subcore_axis_name="s")

    @functools.partial(
        pl.kernel,
        out_type=jax.ShapeDtypeStruct((_NUMI, _PD), jnp.float32),
        mesh=mesh,
        scratch_types=[
            pltpu.VMEM((_BPW,), jnp.int32),
            pltpu.VMEM((_GROW, _PD), jnp.float32),
            pltpu.SemaphoreType.DMA,
        ])
    def k(table_hbm, idx_hbm, out_hbm, idx_v, rows_v, sem):
        wid = jax.lax.axis_index("s") * 2 + jax.lax.axis_index("c")
        base = wid * _BPW
        pltpu.sync_copy(idx_hbm.at[pl.ds(base, _BPW)], idx_v)

        @pl.loop(0, _GCH)
        def _(c):
            off = c * _GROW
            pltpu.async_copy(table_hbm.at[idx_v.at[pl.ds(off, _GROW)]],
                             rows_v, sem).wait()
            pltpu.sync_copy(rows_v, out_hbm.at[pl.ds(base + off, _GROW)])

    return k(statepad, idx_flat)


def _pad_kernel(st_ref, out_ref):
    out_ref[...] = jnp.concatenate(
        [st_ref[...], jnp.zeros((MB, _PD - STATE_DIM), jnp.float32)], axis=1)


def _run_pad(st2d):
    return pl.pallas_call(
        _pad_kernel,
        grid=(B * N // MB,),
        in_specs=[pl.BlockSpec((MB, STATE_DIM), lambda i: (i, 0))],
        out_specs=pl.BlockSpec((MB, _PD), lambda i: (i, 0)),
        out_shape=jax.ShapeDtypeStruct((B * N, _PD), jnp.float32),
        compiler_params=pltpu.CompilerParams(
            dimension_semantics=("parallel",)),
    )(st2d)


@functools.partial(jax.jit, static_argnames=())
def kernel(features, params):
    s_mat, sx, sy = _sample_matrix()
    img_f = jnp.transpose(features.reshape(B, FEAT_DIM, GH * GW), (0, 2, 1))
    st0 = _run_init(s_mat, img_f, params)

    ch0 = sx[None, :] + st0[..., 0] * 0.15
    ch1 = sy[None, :] + st0[..., 1] * 0.15
    base_z = jnp.broadcast_to(params['depth_offset'], (B, N))
    state = jnp.concatenate(
        [ch0[..., None], ch1[..., None], base_z[..., None], st0[..., 3:]],
        axis=-1)

    w1 = params['perc_W1']
    b1 = params['perc_b1'].reshape(1, -1)
    w2, b2 = params['perc_W2'], params['perc_b2'].reshape(1, -1)
    u1, ub1 = params['upd_W1'], params['upd_b1'].reshape(1, -1)
    u2, ub2 = params['upd_W2'], params['upd_b2'].reshape(1, -1)
    step = params['step_size'].reshape(1, 1)

    state2d = state.reshape(B * N, STATE_DIM)
    statepad = _run_pad(state2d)
    for _ in range(N_STEPS):
        pos = state[..., :3]
        posc = jnp.concatenate(
            [pos, jnp.zeros((B, N, 5), jnp.float32)], axis=-1)
        posr = jnp.transpose(posc, (0, 2, 1))
        nbr8 = _run_topk(posc, posr)
        flat_idx = jnp.moveaxis(nbr8[:, :, :K], -1, 0).reshape(_NUMI)
        g3 = _sc_gather(statepad, flat_idx).reshape(K, B * N, _PD)
        state2d, statepad = _run_mlp(state2d, g3, w1, b1, w2, b2, u1, ub1,
                                     u2, ub2, step)
        state = state2d.reshape(B, N, STATE_DIM)

    out = _run_decode(state.reshape(B * N, STATE_DIM))
    return out.reshape(B, N, 14)


# speedup vs baseline: 20.4853x; 1.1329x over previous
import functools

import jax
import jax.numpy as jnp
import numpy as np
from jax.experimental import pallas as pl
from jax.experimental.pallas import tpu as pltpu
from jax.experimental.pallas import tpu_sc as plsc

B = 8
FEAT_DIM = 384
GH = GW = 37
N = 2048
N_STEPS = 8
K = 6
HID = 128
STATE_DIM = 16
RB = 512
MB = 2048


def _dot(a, b, prec=jax.lax.Precision.DEFAULT):
    return jax.lax.dot_general(a, b, (((1,), (0,)), ((), ())),
                               precision=prec,
                               preferred_element_type=jnp.float32)


def _init_kernel(s_ref, img_ref, w1_ref, b1_ref, w2_ref, b2_ref, w3_ref,
                 b3_ref, out_ref):
    samp = _dot(s_ref[...], img_ref[0], jax.lax.Precision.HIGHEST)
    h = jnp.maximum(_dot(samp, w1_ref[...]) + b1_ref[...], 0.0)
    h = jnp.maximum(_dot(h, w2_ref[...]) + b2_ref[...], 0.0)
    out_ref[0] = _dot(h, w3_ref[...]) + b3_ref[...]


def _run_init(s_mat, img_f, params):
    full = lambda shape: pl.BlockSpec(shape, lambda b: (0,) * len(shape))
    return pl.pallas_call(
        _init_kernel,
        grid=(B,),
        in_specs=[
            full((N, GH * GW)),
            pl.BlockSpec((1, GH * GW, FEAT_DIM), lambda b: (b, 0, 0)),
            full((FEAT_DIM, 2 * HID)), full((1, 2 * HID)),
            full((2 * HID, HID)), full((1, HID)),
            full((HID, STATE_DIM)), full((1, STATE_DIM)),
        ],
        out_specs=pl.BlockSpec((1, N, STATE_DIM), lambda b: (b, 0, 0)),
        out_shape=jax.ShapeDtypeStruct((B, N, STATE_DIM), jnp.float32),
        compiler_params=pltpu.CompilerParams(
            dimension_semantics=("parallel",)),
    )(s_mat, img_f,
      params['init_W1'], params['init_b1'].reshape(1, -1),
      params['init_W2'], params['init_b2'].reshape(1, -1),
      params['init_W3'], params['init_b3'].reshape(1, -1))


def _topk_kernel(posc_ref, posr_ref, out_ref):
    pc = posc_ref[0]
    pr = posr_ref[0]
    xc, yc, zc = pc[:, 0:1], pc[:, 1:2], pc[:, 2:3]
    xr, yr, zr = pr[0:1, :], pr[1:2, :], pr[2:3, :]
    sqc = xc * xc + yc * yc + zc * zc
    sqr = xr * xr + yr * yr + zr * zr
    dot = _dot(pc, pr)
    d2 = (sqc + sqr) - 2.0 * dot
    iota_f = jax.lax.broadcasted_iota(jnp.int32, (RB, N), 1).astype(
        jnp.float32)
    boff = pl.program_id(0) * N
    vals = d2
    for t in range(K + 1):
        m = jnp.min(vals, axis=1, keepdims=True)
        idxm = jnp.where(vals == m, iota_f, jnp.float32(N))
        sel = jnp.min(idxm, axis=1, keepdims=True)
        out_ref[0, :, pl.ds(t - 1 if t > 0 else 6, 1)] = (
            sel.astype(jnp.int32) + boff)
        if t < K:
            vals = jnp.where(iota_f == sel, jnp.inf, vals)


def _run_topk(posc, posr):
    return pl.pallas_call(
        _topk_kernel,
        grid=(B, N // RB),
        in_specs=[
            pl.BlockSpec((1, RB, 8), lambda b, r: (b, r, 0)),
            pl.BlockSpec((1, 8, N), lambda b, r: (b, 0, 0)),
        ],
        out_specs=pl.BlockSpec((1, RB, 8), lambda b, r: (b, r, 0)),
        out_shape=jax.ShapeDtypeStruct((B, N, 8), jnp.int32),
        compiler_params=pltpu.CompilerParams(
            dimension_semantics=("parallel", "parallel")),
    )(posc, posr)


def _mlp_kernel(st_ref, g_ref, w1_ref, b1_ref, w2_ref, b2_ref,
                u1_ref, ub1_ref, u2_ref, ub2_ref, step_ref, out_ref, pad_ref):
    st = st_ref[...]
    g = g_ref[...]
    pin = jnp.concatenate(
        [st] + [g[k, :, :STATE_DIM] for k in range(K)], axis=1)
    h = jnp.maximum(_dot(pin, w1_ref[...]) + b1_ref[...], 0.0)
    h = jnp.maximum(_dot(h, w2_ref[...]) + b2_ref[...], 0.0)
    h = jnp.maximum(_dot(h, u1_ref[...]) + ub1_ref[...], 0.0)
    delta = _dot(h, u2_ref[...]) + ub2_ref[...]
    new = st + step_ref[0, 0] * delta
    out_ref[...] = new
    pad_ref[...] = jnp.concatenate(
        [new, jnp.zeros((MB, _PD - STATE_DIM), jnp.float32)], axis=1)


def _run_mlp(st2d, g2d, w1, b1, w2, b2, u1, ub1, u2, ub2, step):
    full = lambda shape: pl.BlockSpec(shape, lambda i: (0,) * len(shape))
    return pl.pallas_call(
        _mlp_kernel,
        grid=(B * N // MB,),
        in_specs=[
            pl.BlockSpec((MB, STATE_DIM), lambda i: (i, 0)),
            pl.BlockSpec((K, MB, _PD), lambda i: (0, i, 0)),
            full(((K + 1) * STATE_DIM, 2 * HID)),
            full((1, 2 * HID)), full((2 * HID, HID)), full((1, HID)),
            full((HID, HID)), full((1, HID)), full((HID, STATE_DIM)),
            full((1, STATE_DIM)), full((1, 1)),
        ],
        out_specs=[pl.BlockSpec((MB, STATE_DIM), lambda i: (i, 0)),
                   pl.BlockSpec((MB, _PD), lambda i: (i, 0))],
        out_shape=[jax.ShapeDtypeStruct((B * N, STATE_DIM), jnp.float32),
                   jax.ShapeDtypeStruct((B * N, _PD), jnp.float32)],
        compiler_params=pltpu.CompilerParams(
            dimension_semantics=("parallel",)),
    )(st2d, g2d, w1, b1, w2, b2, u1, ub1, u2, ub2, step)


def _decode_kernel(st_ref, out_ref):
    st = st_ref[...]
    out_ref[:, 0:3] = st[:, 0:3]
    sc_in = jnp.clip(st[:, 3:6], -10.0, 20.0) + 1.0
    sp = jnp.maximum(sc_in, 0.0) + jnp.log1p(jnp.exp(-jnp.abs(sc_in)))
    out_ref[:, 3:6] = jnp.clip(sp * 0.15, 1e-6, 2.0)
    a1x, a1y, a1z = st[:, 6:7], st[:, 7:8], st[:, 8:9]
    a2x, a2y, a2z = st[:, 9:10], st[:, 10:11], st[:, 11:12]
    n1 = jnp.sqrt(a1x * a1x + a1y * a1y + a1z * a1z) + 1e-8
    b1x, b1y, b1z = a1x / n1, a1y / n1, a1z / n1
    dot = b1x * a2x + b1y * a2y + b1z * a2z
    px, py, pz = a2x - dot * b1x, a2y - dot * b1y, a2z - dot * b1z
    n2 = jnp.sqrt(px * px + py * py + pz * pz) + 1e-8
    b2x, b2y, b2z = px / n2, py / n2, pz / n2
    b3x = b1y * b2z - b1z * b2y
    b3y = b1z * b2x - b1x * b2z
    b3z = b1x * b2y - b1y * b2x
    w = 0.5 * jnp.sqrt(jnp.clip(1.0 + b1x + b2y + b3z, 1e-8))
    out_ref[:, 6:7] = w
    out_ref[:, 7:8] = (b3y - b2z) / (4.0 * w)
    out_ref[:, 8:9] = (b1z - b3x) / (4.0 * w)
    out_ref[:, 9:10] = (b2x - b1y) / (4.0 * w)
    out_ref[:, 10:13] = jax.nn.sigmoid(st[:, 12:15])
    out_ref[:, 13:14] = jax.nn.sigmoid(st[:, 15:16])


def _run_decode(st2d):
    return pl.pallas_call(
        _decode_kernel,
        grid=(B * N // MB,),
        in_specs=[pl.BlockSpec((MB, STATE_DIM), lambda i: (i, 0))],
        out_specs=pl.BlockSpec((MB, 14), lambda i: (i, 0)),
        out_shape=jax.ShapeDtypeStruct((B * N, 14), jnp.float32),
        compiler_params=pltpu.CompilerParams(
            dimension_semantics=("parallel",)),
    )(st2d)


def _sample_matrix():
    i = jnp.arange(N, dtype=jnp.float32)
    ga = np.pi * (3.0 - np.sqrt(5.0))
    r = jnp.sqrt((i + 0.5) / N)
    th = i * ga
    sx = r * jnp.cos(th) * 0.95
    sy = r * jnp.sin(th) * 0.95
    ix = (sx + 1.0) * 0.5 * (GW - 1)
    iy = (sy + 1.0) * 0.5 * (GH - 1)
    x0 = jnp.floor(ix)
    y0 = jnp.floor(iy)
    wx1 = ix - x0
    wy1 = iy - y0
    x0c = jnp.clip(x0, 0, GW - 1).astype(jnp.int32)
    x1c = jnp.clip(x0 + 1, 0, GW - 1).astype(jnp.int32)
    y0c = jnp.clip(y0, 0, GH - 1).astype(jnp.int32)
    y1c = jnp.clip(y0 + 1, 0, GH - 1).astype(jnp.int32)
    hw = GH * GW
    s = (jax.nn.one_hot(y0c * GW + x0c, hw) * ((1 - wy1) * (1 - wx1))[:, None]
         + jax.nn.one_hot(y0c * GW + x1c, hw) * ((1 - wy1) * wx1)[:, None]
         + jax.nn.one_hot(y1c * GW + x0c, hw) * (wy1 * (1 - wx1))[:, None]
         + jax.nn.one_hot(y1c * GW + x1c, hw) * (wy1 * wx1)[:, None])
    return s, sx, sy


_NUMI = B * N * K


_NW = 32
_BPW = _NUMI // _NW
_GCH = 4
_GROW = _BPW // _GCH
_PD = 128


def _sc_gather(statepad, idx_flat):
    mesh = plsc.VectorSubcoreMesh(core_axis_name="c", subcore_axis_name="s")

    @functools.partial(
        pl.kernel,
        out_type=jax.ShapeDtypeStruct((_NUMI, _PD), jnp.float32),
        mesh=mesh,
        scratch_types=[
            pltpu.VMEM((_BPW,), jnp.int32),
            pltpu.VMEM((_GROW, _PD), jnp.float32),
            pltpu.SemaphoreType.DMA,
        ])
    def k(table_hbm, idx_hbm, out_hbm, idx_v, rows_v, sem):
        wid = jax.lax.axis_index("s") * 2 + jax.lax.axis_index("c")
        base = wid * _BPW
        pltpu.sync_copy(idx_hbm.at[pl.ds(base, _BPW)], idx_v)

        @pl.loop(0, _GCH)
        def _(c):
            off = c * _GROW
            pltpu.async_copy(table_hbm.at[idx_v.at[pl.ds(off, _GROW)]],
                             rows_v, sem).wait()
            pltpu.sync_copy(rows_v, out_hbm.at[pl.ds(base + off, _GROW)])

    return k(statepad, idx_flat)


def _pad_kernel(st_ref, out_ref):
    out_ref[...] = jnp.concatenate(
        [st_ref[...], jnp.zeros((MB, _PD - STATE_DIM), jnp.float32)], axis=1)


def _run_pad(st2d):
    return pl.pallas_call(
        _pad_kernel,
        grid=(B * N // MB,),
        in_specs=[pl.BlockSpec((MB, STATE_DIM), lambda i: (i, 0))],
        out_specs=pl.BlockSpec((MB, _PD), lambda i: (i, 0)),
        out_shape=jax.ShapeDtypeStruct((B * N, _PD), jnp.float32),
        compiler_params=pltpu.CompilerParams(
            dimension_semantics=("parallel",)),
    )(st2d)


@functools.partial(jax.jit, static_argnames=())
def kernel(features, params):
    s_mat, sx, sy = _sample_matrix()
    img_f = jnp.transpose(features.reshape(B, FEAT_DIM, GH * GW), (0, 2, 1))
    st0 = _run_init(s_mat, img_f, params)

    ch0 = sx[None, :] + st0[..., 0] * 0.15
    ch1 = sy[None, :] + st0[..., 1] * 0.15
    base_z = jnp.broadcast_to(params['depth_offset'], (B, N))
    state = jnp.concatenate(
        [ch0[..., None], ch1[..., None], base_z[..., None], st0[..., 3:]],
        axis=-1)

    w1 = params['perc_W1']
    b1 = params['perc_b1'].reshape(1, -1)
    w2, b2 = params['perc_W2'], params['perc_b2'].reshape(1, -1)
    u1, ub1 = params['upd_W1'], params['upd_b1'].reshape(1, -1)
    u2, ub2 = params['upd_W2'], params['upd_b2'].reshape(1, -1)
    step = params['step_size'].reshape(1, 1)

    state2d = state.reshape(B * N, STATE_DIM)
    statepad = _run_pad(state2d)
    for _ in range(N_STEPS):
        pos = state[..., :3]
        posc = jnp.concatenate(
            [pos, jnp.zeros((B, N, 5), jnp.float32)], axis=-1)
        posr = jnp.transpose(posc, (0, 2, 1))
        nbr8 = _run_topk(posc, posr)
        flat_idx = jnp.moveaxis(nbr8[:, :, :K], -1, 0).reshape(_NUMI)
        g3 = _sc_gather(statepad, flat_idx).reshape(K, B * N, _PD)
        state2d, statepad = _run_mlp(state2d, g3, w1, b1, w2, b2, u1, ub1,
                                     u2, ub2, step)
        state = state2d.reshape(B, N, STATE_DIM)

    out = _run_decode(state.reshape(B * N, STATE_DIM))
    return out.reshape(B, N, 14)


# speedup vs baseline: 21.1574x; 1.0328x over previous
import functools

import jax
import jax.numpy as jnp
import numpy as np
from jax.experimental import pallas as pl
from jax.experimental.pallas import tpu as pltpu
from jax.experimental.pallas import tpu_sc as plsc

B = 8
FEAT_DIM = 384
GH = GW = 37
N = 2048
N_STEPS = 8
K = 6
HID = 128
STATE_DIM = 16
RB = 512
MB = 2048


def _dot(a, b, prec=jax.lax.Precision.DEFAULT):
    return jax.lax.dot_general(a, b, (((1,), (0,)), ((), ())),
                               precision=prec,
                               preferred_element_type=jnp.float32)


def _init_kernel(s_ref, img_ref, w1_ref, b1_ref, w2_ref, b2_ref, w3_ref,
                 b3_ref, out_ref):
    samp = _dot(s_ref[...], img_ref[0], jax.lax.Precision.HIGHEST)
    h = jnp.maximum(_dot(samp, w1_ref[...]) + b1_ref[...], 0.0)
    h = jnp.maximum(_dot(h, w2_ref[...]) + b2_ref[...], 0.0)
    out_ref[0] = _dot(h, w3_ref[...]) + b3_ref[...]


def _run_init(s_mat, img_f, params):
    full = lambda shape: pl.BlockSpec(shape, lambda b: (0,) * len(shape))
    return pl.pallas_call(
        _init_kernel,
        grid=(B,),
        in_specs=[
            full((N, GH * GW)),
            pl.BlockSpec((1, GH * GW, FEAT_DIM), lambda b: (b, 0, 0)),
            full((FEAT_DIM, 2 * HID)), full((1, 2 * HID)),
            full((2 * HID, HID)), full((1, HID)),
            full((HID, STATE_DIM)), full((1, STATE_DIM)),
        ],
        out_specs=pl.BlockSpec((1, N, STATE_DIM), lambda b: (b, 0, 0)),
        out_shape=jax.ShapeDtypeStruct((B, N, STATE_DIM), jnp.float32),
        compiler_params=pltpu.CompilerParams(
            dimension_semantics=("parallel",)),
    )(s_mat, img_f,
      params['init_W1'], params['init_b1'].reshape(1, -1),
      params['init_W2'], params['init_b2'].reshape(1, -1),
      params['init_W3'], params['init_b3'].reshape(1, -1))


def _topk_kernel(st_ref, stt_ref, out_ref):
    st = st_ref[0]
    stt = stt_ref[0]
    lane = jax.lax.broadcasted_iota(jnp.int32, (RB, STATE_DIM), 1)
    pc = jnp.where(lane < 3, st, 0.0)
    row = jax.lax.broadcasted_iota(jnp.int32, (STATE_DIM, N), 0)
    pr = jnp.where(row < 3, stt, 0.0)
    xc, yc, zc = pc[:, 0:1], pc[:, 1:2], pc[:, 2:3]
    xr, yr, zr = pr[0:1, :], pr[1:2, :], pr[2:3, :]
    sqc = xc * xc + yc * yc + zc * zc
    sqr = xr * xr + yr * yr + zr * zr
    dot = _dot(pc, pr)
    d2 = (sqc + sqr) - 2.0 * dot
    iota_f = jax.lax.broadcasted_iota(jnp.int32, (RB, N), 1).astype(
        jnp.float32)
    boff = pl.program_id(0) * N
    vals = d2
    for t in range(K + 1):
        m = jnp.min(vals, axis=1, keepdims=True)
        idxm = jnp.where(vals == m, iota_f, jnp.float32(N))
        sel = jnp.min(idxm, axis=1, keepdims=True)
        out_ref[0, :, pl.ds(t - 1 if t > 0 else 6, 1)] = (
            sel.astype(jnp.int32) + boff)
        if t < K:
            vals = jnp.where(iota_f == sel, jnp.inf, vals)


def _run_topk(state, stateT):
    return pl.pallas_call(
        _topk_kernel,
        grid=(B, N // RB),
        in_specs=[
            pl.BlockSpec((1, RB, STATE_DIM), lambda b, r: (b, r, 0)),
            pl.BlockSpec((1, STATE_DIM, N), lambda b, r: (b, 0, 0)),
        ],
        out_specs=pl.BlockSpec((1, RB, 8), lambda b, r: (b, r, 0)),
        out_shape=jax.ShapeDtypeStruct((B, N, 8), jnp.int32),
        compiler_params=pltpu.CompilerParams(
            dimension_semantics=("parallel", "parallel")),
    )(state, stateT)


def _mlp_kernel(st_ref, g_ref, w1_ref, b1_ref, w2_ref, b2_ref,
                u1_ref, ub1_ref, u2_ref, ub2_ref, step_ref, out_ref, pad_ref,
                tr_ref):
    st = st_ref[...]
    g = g_ref[...]
    pin = jnp.concatenate(
        [st] + [g[k, :, :STATE_DIM] for k in range(K)], axis=1)
    h = jnp.maximum(_dot(pin, w1_ref[...]) + b1_ref[...], 0.0)
    h = jnp.maximum(_dot(h, w2_ref[...]) + b2_ref[...], 0.0)
    h = jnp.maximum(_dot(h, u1_ref[...]) + ub1_ref[...], 0.0)
    delta = _dot(h, u2_ref[...]) + ub2_ref[...]
    new = st + step_ref[0, 0] * delta
    out_ref[...] = new
    pad_ref[...] = jnp.concatenate(
        [new, jnp.zeros((MB, _PD - STATE_DIM), jnp.float32)], axis=1)
    tr_ref[0] = new.T


def _run_mlp(st2d, g2d, w1, b1, w2, b2, u1, ub1, u2, ub2, step):
    full = lambda shape: pl.BlockSpec(shape, lambda i: (0,) * len(shape))
    return pl.pallas_call(
        _mlp_kernel,
        grid=(B * N // MB,),
        in_specs=[
            pl.BlockSpec((MB, STATE_DIM), lambda i: (i, 0)),
            pl.BlockSpec((K, MB, _PD), lambda i: (0, i, 0)),
            full(((K + 1) * STATE_DIM, 2 * HID)),
            full((1, 2 * HID)), full((2 * HID, HID)), full((1, HID)),
            full((HID, HID)), full((1, HID)), full((HID, STATE_DIM)),
            full((1, STATE_DIM)), full((1, 1)),
        ],
        out_specs=[pl.BlockSpec((MB, STATE_DIM), lambda i: (i, 0)),
                   pl.BlockSpec((MB, _PD), lambda i: (i, 0)),
                   pl.BlockSpec((1, STATE_DIM, MB), lambda i: (i, 0, 0))],
        out_shape=[jax.ShapeDtypeStruct((B * N, STATE_DIM), jnp.float32),
                   jax.ShapeDtypeStruct((B * N, _PD), jnp.float32),
                   jax.ShapeDtypeStruct((B, STATE_DIM, N), jnp.float32)],
        compiler_params=pltpu.CompilerParams(
            dimension_semantics=("parallel",)),
    )(st2d, g2d, w1, b1, w2, b2, u1, ub1, u2, ub2, step)


def _decode_kernel(st_ref, out_ref):
    st = st_ref[...]
    out_ref[:, 0:3] = st[:, 0:3]
    sc_in = jnp.clip(st[:, 3:6], -10.0, 20.0) + 1.0
    sp = jnp.maximum(sc_in, 0.0) + jnp.log1p(jnp.exp(-jnp.abs(sc_in)))
    out_ref[:, 3:6] = jnp.clip(sp * 0.15, 1e-6, 2.0)
    a1x, a1y, a1z = st[:, 6:7], st[:, 7:8], st[:, 8:9]
    a2x, a2y, a2z = st[:, 9:10], st[:, 10:11], st[:, 11:12]
    n1 = jnp.sqrt(a1x * a1x + a1y * a1y + a1z * a1z) + 1e-8
    b1x, b1y, b1z = a1x / n1, a1y / n1, a1z / n1
    dot = b1x * a2x + b1y * a2y + b1z * a2z
    px, py, pz = a2x - dot * b1x, a2y - dot * b1y, a2z - dot * b1z
    n2 = jnp.sqrt(px * px + py * py + pz * pz) + 1e-8
    b2x, b2y, b2z = px / n2, py / n2, pz / n2
    b3x = b1y * b2z - b1z * b2y
    b3y = b1z * b2x - b1x * b2z
    b3z = b1x * b2y - b1y * b2x
    w = 0.5 * jnp.sqrt(jnp.clip(1.0 + b1x + b2y + b3z, 1e-8))
    out_ref[:, 6:7] = w
    out_ref[:, 7:8] = (b3y - b2z) / (4.0 * w)
    out_ref[:, 8:9] = (b1z - b3x) / (4.0 * w)
    out_ref[:, 9:10] = (b2x - b1y) / (4.0 * w)
    out_ref[:, 10:13] = jax.nn.sigmoid(st[:, 12:15])
    out_ref[:, 13:14] = jax.nn.sigmoid(st[:, 15:16])


def _run_decode(st2d):
    return pl.pallas_call(
        _decode_kernel,
        grid=(B * N // MB,),
        in_specs=[pl.BlockSpec((MB, STATE_DIM), lambda i: (i, 0))],
        out_specs=pl.BlockSpec((MB, 14), lambda i: (i, 0)),
        out_shape=jax.ShapeDtypeStruct((B * N, 14), jnp.float32),
        compiler_params=pltpu.CompilerParams(
            dimension_semantics=("parallel",)),
    )(st2d)


def _sample_matrix():
    i = jnp.arange(N, dtype=jnp.float32)
    ga = np.pi * (3.0 - np.sqrt(5.0))
    r = jnp.sqrt((i + 0.5) / N)
    th = i * ga
    sx = r * jnp.cos(th) * 0.95
    sy = r * jnp.sin(th) * 0.95
    ix = (sx + 1.0) * 0.5 * (GW - 1)
    iy = (sy + 1.0) * 0.5 * (GH - 1)
    x0 = jnp.floor(ix)
    y0 = jnp.floor(iy)
    wx1 = ix - x0
    wy1 = iy - y0
    x0c = jnp.clip(x0, 0, GW - 1).astype(jnp.int32)
    x1c = jnp.clip(x0 + 1, 0, GW - 1).astype(jnp.int32)
    y0c = jnp.clip(y0, 0, GH - 1).astype(jnp.int32)
    y1c = jnp.clip(y0 + 1, 0, GH - 1).astype(jnp.int32)
    hw = GH * GW
    s = (jax.nn.one_hot(y0c * GW + x0c, hw) * ((1 - wy1) * (1 - wx1))[:, None]
         + jax.nn.one_hot(y0c * GW + x1c, hw) * ((1 - wy1) * wx1)[:, None]
         + jax.nn.one_hot(y1c * GW + x0c, hw) * (wy1 * (1 - wx1))[:, None]
         + jax.nn.one_hot(y1c * GW + x1c, hw) * (wy1 * wx1)[:, None])
    return s, sx, sy


_NUMI = B * N * K


_NW = 32
_BPW = _NUMI // _NW
_GCH = 4
_GROW = _BPW // _GCH
_PD = 128


def _sc_gather(statepad, idx_flat):
    mesh = plsc.VectorSubcoreMesh(core_axis_name="c", subcore_axis_name="s")

    @functools.partial(
        pl.kernel,
        out_type=jax.ShapeDtypeStruct((_NUMI, _PD), jnp.float32),
        mesh=mesh,
        scratch_types=[
            pltpu.VMEM((_BPW,), jnp.int32),
            pltpu.VMEM((_GROW, _PD), jnp.float32),
            pltpu.SemaphoreType.DMA,
        ])
    def k(table_hbm, idx_hbm, out_hbm, idx_v, rows_v, sem):
        wid = jax.lax.axis_index("s") * 2 + jax.lax.axis_index("c")
        base = wid * _BPW
        pltpu.sync_copy(idx_hbm.at[pl.ds(base, _BPW)], idx_v)

        @pl.loop(0, _GCH)
        def _(c):
            off = c * _GROW
            pltpu.async_copy(table_hbm.at[idx_v.at[pl.ds(off, _GROW)]],
                             rows_v, sem).wait()
            pltpu.sync_copy(rows_v, out_hbm.at[pl.ds(base + off, _GROW)])

    return k(statepad, idx_flat)


def _pad_kernel(st_ref, out_ref):
    out_ref[...] = jnp.concatenate(
        [st_ref[...], jnp.zeros((MB, _PD - STATE_DIM), jnp.float32)], axis=1)


def _run_pad(st2d):
    return pl.pallas_call(
        _pad_kernel,
        grid=(B * N // MB,),
        in_specs=[pl.BlockSpec((MB, STATE_DIM), lambda i: (i, 0))],
        out_specs=pl.BlockSpec((MB, _PD), lambda i: (i, 0)),
        out_shape=jax.ShapeDtypeStruct((B * N, _PD), jnp.float32),
        compiler_params=pltpu.CompilerParams(
            dimension_semantics=("parallel",)),
    )(st2d)


@functools.partial(jax.jit, static_argnames=())
def kernel(features, params):
    s_mat, sx, sy = _sample_matrix()
    img_f = jnp.transpose(features.reshape(B, FEAT_DIM, GH * GW), (0, 2, 1))
    st0 = _run_init(s_mat, img_f, params)

    ch0 = sx[None, :] + st0[..., 0] * 0.15
    ch1 = sy[None, :] + st0[..., 1] * 0.15
    base_z = jnp.broadcast_to(params['depth_offset'], (B, N))
    state = jnp.concatenate(
        [ch0[..., None], ch1[..., None], base_z[..., None], st0[..., 3:]],
        axis=-1)

    w1 = params['perc_W1']
    b1 = params['perc_b1'].reshape(1, -1)
    w2, b2 = params['perc_W2'], params['perc_b2'].reshape(1, -1)
    u1, ub1 = params['upd_W1'], params['upd_b1'].reshape(1, -1)
    u2, ub2 = params['upd_W2'], params['upd_b2'].reshape(1, -1)
    step = params['step_size'].reshape(1, 1)

    state2d = state.reshape(B * N, STATE_DIM)
    statepad = _run_pad(state2d)
    stateT = jnp.transpose(state, (0, 2, 1))
    state3 = state
    for _ in range(N_STEPS):
        nbr8 = _run_topk(state3, stateT)
        flat_idx = jnp.moveaxis(nbr8[:, :, :K], -1, 0).reshape(_NUMI)
        g3 = _sc_gather(statepad, flat_idx).reshape(K, B * N, _PD)
        state2d, statepad, stateT = _run_mlp(state2d, g3, w1, b1, w2, b2,
                                             u1, ub1, u2, ub2, step)
        state3 = state2d.reshape(B, N, STATE_DIM)

    out = _run_decode(state2d)
    return out.reshape(B, N, 14)
